# baseline (speedup 1.0000x reference)
import functools

import jax
import jax.numpy as jnp
import numpy as np
from jax import lax
from jax.experimental import pallas as pl
from jax.experimental.pallas import tpu as pltpu
from jax.experimental.pallas import tpu_sc as plsc

_DIAGONAL_OFFSET = 2

_B = 4
_UT_LEN = 130305
_D = 64
_BH = 2
_ROW = _BH * _D

_SEG = 128
_NFULL = _UT_LEN // _SEG
_LAST_BASE = _UT_LEN - _SEG
_NW = 32
_GPW = 32
_NIDX = 1024


def _reverse_perm(ut_len, diagonal_offset):
    seq_len = int(np.sqrt(2 * ut_len + 0.25) - 0.5) + diagonal_offset
    ut_indexes = np.triu_indices(seq_len, diagonal_offset)
    assert len(ut_indexes[0]) == ut_len
    mat_ut_indexes = np.zeros(shape=(seq_len, seq_len), dtype="int")
    mat_ut_indexes[ut_indexes] = np.arange(ut_len)
    mask_ut = np.zeros(shape=(seq_len, seq_len), dtype="bool")
    mask_ut[ut_indexes] = True
    mat_indexes = mat_ut_indexes + np.multiply(~mask_ut, mat_ut_indexes.T)
    mat_rc_indexes = mat_indexes[::-1, ::-1]
    return mat_rc_indexes[ut_indexes]


@functools.lru_cache(maxsize=None)
def _index_planes():
    perm = _reverse_perm(_UT_LEN, _DIAGONAL_OFFSET).astype(np.int32)
    iota = np.arange(_UT_LEN, dtype=np.int32)
    planes = np.zeros((2, _NIDX, _SEG), dtype=np.int32)
    for p, src in ((0, iota), (1, perm)):
        planes[p, :_NFULL] = src[: _NFULL * _SEG].reshape(_NFULL, _SEG)
        planes[p, _NFULL] = src[_LAST_BASE:]
    return planes


def _sc_body(x_hbm, idx_hbm, rev_hbm, out_hbm, idx_v, dst_v, rows_v, rev_v, sem):
    nc = lax.axis_size("c")
    wid = lax.axis_index("s") * nc + lax.axis_index("c")

    pltpu.sync_copy(rev_hbm, rev_v)
    rev = rev_v[...][0]
    plane = jnp.where(rev != 0, 1, 0)

    base_g = wid * _GPW
    nt = jnp.minimum(_GPW, _NFULL - base_g)

    pltpu.sync_copy(idx_hbm.at[plane, pl.ds(base_g, _GPW)], idx_v)

    def body(t, carry):
        pltpu.async_copy(x_hbm.at[idx_v.at[t]], rows_v, sem).wait()
        pltpu.sync_copy(rows_v, out_hbm.at[pl.ds((base_g + t) * _SEG, _SEG)])
        return carry

    lax.fori_loop(0, nt, body, 0)

    @pl.when(wid == _NW - 1)
    def _():
        t_last = _NFULL - (_NW - 1) * _GPW
        dst_base = (_NFULL // 8) * 8
        pltpu.sync_copy(idx_hbm.at[0, pl.ds(dst_base, 8)], dst_v)
        pltpu.async_copy(x_hbm.at[idx_v.at[t_last]], rows_v, sem).wait()
        pltpu.async_copy(rows_v, out_hbm.at[dst_v.at[_NFULL - dst_base]], sem).wait()


@jax.jit
def _sc_gather(xt, idx, rev_vec):
    call = pl.kernel(
        _sc_body,
        out_type=jax.ShapeDtypeStruct((_UT_LEN, _ROW), jnp.float32),
        mesh=plsc.VectorSubcoreMesh(core_axis_name="c", subcore_axis_name="s"),
        scratch_types=[
            pltpu.VMEM((_GPW, _SEG), jnp.int32),
            pltpu.VMEM((8, _SEG), jnp.int32),
            pltpu.VMEM((_SEG, _ROW), jnp.float32),
            pltpu.VMEM((16,), jnp.int32),
            pltpu.SemaphoreType.DMA,
        ],
    )
    return call(xt, idx, rev_vec)


def kernel(x_ut, reverse):
    assert x_ut.shape == (_B, _UT_LEN, _D), x_ut.shape
    idx = jnp.asarray(_index_planes())
    rev_vec = jnp.broadcast_to(jnp.asarray(reverse, jnp.int32), (16,))
    halves = []
    for h in range(_B // _BH):
        xt = jnp.transpose(x_ut[h * _BH:(h + 1) * _BH], (1, 0, 2))
        xt = xt.reshape(_UT_LEN, _ROW)
        out = _sc_gather(xt, idx, rev_vec)
        halves.append(jnp.transpose(out.reshape(_UT_LEN, _BH, _D), (1, 0, 2)))
    return jnp.concatenate(halves, axis=0)


# speedup vs baseline: 1.4540x; 1.4540x over previous
import functools

import jax
import jax.numpy as jnp
import numpy as np
from jax import lax
from jax.experimental import pallas as pl
from jax.experimental.pallas import tpu as pltpu
from jax.experimental.pallas import tpu_sc as plsc

_DIAGONAL_OFFSET = 2

_B = 4
_UT_LEN = 130305
_D = 64
_ROW = _B * _D

_SEG = 128
_NFULL = _UT_LEN // _SEG
_LAST_BASE = _UT_LEN - _SEG
_NW = 32
_GPW = 32
_NIDX = 1024


def _reverse_perm(ut_len, diagonal_offset):
    seq_len = int(np.sqrt(2 * ut_len + 0.25) - 0.5) + diagonal_offset
    ut_indexes = np.triu_indices(seq_len, diagonal_offset)
    assert len(ut_indexes[0]) == ut_len
    mat_ut_indexes = np.zeros(shape=(seq_len, seq_len), dtype="int")
    mat_ut_indexes[ut_indexes] = np.arange(ut_len)
    mask_ut = np.zeros(shape=(seq_len, seq_len), dtype="bool")
    mask_ut[ut_indexes] = True
    mat_indexes = mat_ut_indexes + np.multiply(~mask_ut, mat_ut_indexes.T)
    mat_rc_indexes = mat_indexes[::-1, ::-1]
    return mat_rc_indexes[ut_indexes]


@functools.lru_cache(maxsize=None)
def _index_planes():
    perm = _reverse_perm(_UT_LEN, _DIAGONAL_OFFSET).astype(np.int32)
    iota = np.arange(_UT_LEN, dtype=np.int32)
    planes = np.zeros((2, _NIDX, _SEG), dtype=np.int32)
    for p, src in ((0, iota), (1, perm)):
        planes[p, :_NFULL] = src[: _NFULL * _SEG].reshape(_NFULL, _SEG)
        planes[p, _NFULL] = src[_LAST_BASE:]
    return planes


def _sc_body(x_hbm, idx_hbm, rev_hbm, out_hbm,
             idx_v, dst_v, rows0, rows1, rev_v, sem_a, sem_b, sem_st):
    nc = lax.axis_size("c")
    wid = lax.axis_index("s") * nc + lax.axis_index("c")

    pltpu.sync_copy(rev_hbm, rev_v)
    rev = rev_v[...][0]
    plane = jnp.where(rev != 0, 1, 0)

    base_g = wid * _GPW
    nt = jnp.minimum(_GPW, _NFULL - base_g)
    ntp = nt // 2

    pltpu.sync_copy(idx_hbm.at[plane, pl.ds(base_g, _GPW)], idx_v)

    def start_gather(t, buf, sem):
        return pltpu.async_copy(x_hbm.at[idx_v.at[t]], buf, sem)

    def wait_gather(t, buf, sem):
        pltpu.make_async_copy(x_hbm.at[idx_v.at[t]], buf, sem).wait()

    def fire_store(t, buf):
        pltpu.async_copy(buf, out_hbm.at[pl.ds((base_g + t) * _SEG, _SEG)],
                         sem_st)

    def drain_store():
        pltpu.make_async_copy(x_hbm.at[pl.ds(0, _SEG)], rows0, sem_st).wait()

    start_gather(0, rows0, sem_a)

    def pair_body(t2, carry):
        a = 2 * t2

        @pl.when(t2 > 0)
        def _():
            drain_store()

        start_gather(a + 1, rows1, sem_b)
        wait_gather(a, rows0, sem_a)
        fire_store(a, rows0)

        @pl.when(t2 + 1 < ntp)
        def _():
            drain_store()
            start_gather(a + 2, rows0, sem_a)

        wait_gather(a + 1, rows1, sem_b)
        fire_store(a + 1, rows1)
        return carry

    lax.fori_loop(0, ntp, pair_body, 0)
    drain_store()
    drain_store()

    @pl.when(wid == _NW - 1)
    def _():
        t_last = _NFULL - (_NW - 1) * _GPW
        dst_base = (_NFULL // 8) * 8
        pltpu.sync_copy(idx_hbm.at[0, pl.ds(dst_base, 8)], dst_v)
        pltpu.async_copy(x_hbm.at[idx_v.at[t_last]], rows0, sem_a).wait()
        pltpu.async_copy(rows0, out_hbm.at[dst_v.at[_NFULL - dst_base]],
                         sem_a).wait()


@jax.jit
def _sc_gather(xt, idx, rev_vec):
    call = pl.kernel(
        _sc_body,
        out_type=jax.ShapeDtypeStruct((_UT_LEN, _ROW), jnp.float32),
        mesh=plsc.VectorSubcoreMesh(core_axis_name="c", subcore_axis_name="s"),
        scratch_types=[
            pltpu.VMEM((_GPW, _SEG), jnp.int32),
            pltpu.VMEM((8, _SEG), jnp.int32),
            pltpu.VMEM((_SEG, _ROW), jnp.float32),
            pltpu.VMEM((_SEG, _ROW), jnp.float32),
            pltpu.VMEM((16,), jnp.int32),
            pltpu.SemaphoreType.DMA,
            pltpu.SemaphoreType.DMA,
            pltpu.SemaphoreType.DMA,
        ],
    )
    return call(xt, idx, rev_vec)


def kernel(x_ut, reverse):
    assert x_ut.shape == (_B, _UT_LEN, _D), x_ut.shape
    xt = jnp.transpose(x_ut, (1, 0, 2)).reshape(_UT_LEN, _ROW)
    idx = jnp.asarray(_index_planes())
    rev_vec = jnp.broadcast_to(jnp.asarray(reverse, jnp.int32), (16,))
    out = _sc_gather(xt, idx, rev_vec)
    return jnp.transpose(out.reshape(_UT_LEN, _B, _D), (1, 0, 2))
